# Initial kernel scaffold; baseline (speedup 1.0000x reference)
import jax
import jax.numpy as jnp
from jax.experimental import pallas as pl


def kernel(edge_index, edge_values, ego_embeddings, W, b):
    raise NotImplementedError("write your pallas kernel here")


# speedup vs baseline: 6.2506x; 6.2506x over previous
import functools

import jax
import jax.numpy as jnp
from jax import lax
from jax.experimental import pallas as pl
from jax.experimental.pallas import tpu as pltpu
from jax.experimental.pallas import tpu_sc as plsc

_N = 10000
_E = 320000
_D = 128

_NC = 2
_NS = 16
_NW = _NC * _NS
_EPW = _E // _NW
_CH = 80
_NCHUNK = _EPW // _CH
_G = _CH // 16
_RPW = 624
_TAIL = _N - _NS * _RPW

_mesh = plsc.VectorSubcoreMesh(core_axis_name="c", subcore_axis_name="s")


@functools.partial(
    pl.kernel,
    out_type=jax.ShapeDtypeStruct((_NC, _N, _D), jnp.float32),
    mesh=_mesh,
    scratch_types=[
        pltpu.VMEM_SHARED((_N, _D), jnp.float32),
        pltpu.VMEM((_CH,), jnp.int32),
        pltpu.VMEM((_CH,), jnp.int32),
        pltpu.VMEM((_CH,), jnp.int32),
        pltpu.VMEM((_CH,), jnp.int32),
        pltpu.VMEM((_CH,), jnp.float32),
        pltpu.VMEM((_CH,), jnp.float32),
        pltpu.VMEM((_CH, _D), jnp.float32),
        pltpu.VMEM((_CH, _D), jnp.float32),
        pltpu.SemaphoreType.DMA,
        pltpu.SemaphoreType.DMA,
    ],
)
def _sc_aggregate(src_hbm, dst_hbm, ev_hbm, ego_hbm, zeros_hbm, out_hbm,
                  acc, srcb0, srcb1, dstb0, dstb1, evb0, evb1,
                  rows0, rows1, sem0, sem1):
    c = lax.axis_index("c")
    s = lax.axis_index("s")
    wid = s * _NC + c

    rbase = s * _RPW

    def init_slice(base, n):
        @pl.when(c == 0)
        def _():
            pltpu.sync_copy(ego_hbm.at[pl.ds(base, n)],
                            acc.at[pl.ds(base, n)])

        @pl.when(c != 0)
        def _():
            pltpu.sync_copy(zeros_hbm.at[pl.ds(base, n)],
                            acc.at[pl.ds(base, n)])

    init_slice(rbase, _RPW)

    @pl.when(s == 0)
    def _():
        init_slice(_NS * _RPW, _TAIL)

    plsc.subcore_barrier()

    sems = (sem0, sem1)
    srcbs = (srcb0, srcb1)
    dstbs = (dstb0, dstb1)
    evbs = (evb0, evb1)
    rowss = (rows0, rows1)
    handles = [None, None]

    ebase = wid * _EPW

    def load_idx(i, b):
        sl = pl.ds(ebase + i * _CH, _CH)
        pltpu.sync_copy(src_hbm.at[sl], srcbs[b])
        pltpu.sync_copy(dst_hbm.at[sl], dstbs[b])
        pltpu.sync_copy(ev_hbm.at[sl], evbs[b])

    def start_gather(b):
        handles[b] = pltpu.make_async_copy(
            ego_hbm.at[srcbs[b]], rowss[b], sems[b])
        handles[b].start()

    def scale_chunk(b):
        rr = rowss[b]
        evr = evbs[b]

        def body(g, carry):
            wvec = evr[pl.ds(g * 16, 16)]
            base = g * 16
            for l in range(16):
                w = lax.gather(
                    wvec, jnp.full((16, 1), l, jnp.int32),
                    lax.GatherDimensionNumbers(
                        offset_dims=(), collapsed_slice_dims=(0,),
                        start_index_map=(0,)),
                    (1,), mode=lax.GatherScatterMode.PROMISE_IN_BOUNDS)
                e = base + l
                for j in range(_D // 16):
                    sl = pl.ds(j * 16, 16)
                    rr[e, sl] = rr[e, sl] * w
            return carry

        lax.fori_loop(0, _G, body, 0)

    def wait_gather(b):
        pltpu.make_async_copy(ego_hbm.at[srcbs[b]], rowss[b], sems[b]).wait()

    def one_chunk(i, b, last):
        if not last:
            start_gather(b ^ 1)
        wait_gather(b)
        scale_chunk(b)
        pltpu.sync_copy(rowss[b], acc.at[dstbs[b]], add=True)
        if not last:
            @pl.when(i + 2 < _NCHUNK)
            def _():
                load_idx(i + 2, b)

    load_idx(0, 0)
    load_idx(1, 1)
    start_gather(0)

    def pair_body(k, carry):
        one_chunk(2 * k, 0, False)
        one_chunk(2 * k + 1, 1, False)
        return carry

    lax.fori_loop(0, (_NCHUNK - 1) // 2, pair_body, 0)
    one_chunk(_NCHUNK - 1, (_NCHUNK - 1) & 1, True)

    plsc.subcore_barrier()

    pltpu.sync_copy(acc.at[pl.ds(rbase, _RPW)],
                    out_hbm.at[c, pl.ds(rbase, _RPW)])

    @pl.when(s == 0)
    def _():
        pltpu.sync_copy(acc.at[pl.ds(_NS * _RPW, _TAIL)],
                        out_hbm.at[c, pl.ds(_NS * _RPW, _TAIL)])


def _tc_body(p_ref, wt_ref, b_ref, o_ref):
    x = p_ref[0] + p_ref[1]
    y = jnp.dot(x, wt_ref[...], preferred_element_type=jnp.float32)
    y = y + b_ref[...]
    o_ref[...] = jnp.where(y >= 0, y, y * jnp.float32(0.01))


_BR = 1000


def _tc_dense(partial, wt, b2):
    return pl.pallas_call(
        _tc_body,
        grid=(_N // _BR,),
        in_specs=[
            pl.BlockSpec((_NC, _BR, _D), lambda i: (0, i, 0)),
            pl.BlockSpec((_D, _D), lambda i: (0, 0)),
            pl.BlockSpec((1, _D), lambda i: (0, 0)),
        ],
        out_specs=pl.BlockSpec((_BR, _D), lambda i: (i, 0)),
        out_shape=jax.ShapeDtypeStruct((_N, _D), jnp.float32),
    )(partial, wt, b2)


@jax.jit
def kernel(edge_index, edge_values, ego_embeddings, W, b):
    src = edge_index[0]
    dst = edge_index[1]
    ev = edge_values
    zeros = jnp.zeros((_N, _D), jnp.float32)
    partial = _sc_aggregate(src, dst, ev, ego_embeddings, zeros)
    return _tc_dense(partial, W.T, b.reshape(1, _D))


# speedup vs baseline: 9.9310x; 1.5888x over previous
import functools

import jax
import jax.numpy as jnp
from jax import lax
from jax.experimental import pallas as pl
from jax.experimental.pallas import tpu as pltpu
from jax.experimental.pallas import tpu_sc as plsc

_N = 10000
_E = 320000
_D = 128

_NC = 2
_NS = 16
_NW = _NC * _NS
_EPW = _E // _NW
_CH = 80
_NCHUNK = _EPW // _CH
_G = _CH // 16
_RPW = 624
_TAIL = _N - _NS * _RPW

_mesh = plsc.VectorSubcoreMesh(core_axis_name="c", subcore_axis_name="s")


@functools.partial(
    pl.kernel,
    out_type=jax.ShapeDtypeStruct((_NC, _N, _D), jnp.float32),
    mesh=_mesh,
    scratch_types=[
        pltpu.VMEM_SHARED((_N, _D), jnp.float32),
        pltpu.VMEM((_CH,), jnp.int32),
        pltpu.VMEM((_CH,), jnp.int32),
        pltpu.VMEM((_CH,), jnp.int32),
        pltpu.VMEM((_CH,), jnp.int32),
        pltpu.VMEM((_CH,), jnp.float32),
        pltpu.VMEM((_CH,), jnp.float32),
        pltpu.VMEM((_CH, _D), jnp.float32),
        pltpu.VMEM((_CH, _D), jnp.float32),
        pltpu.VMEM((_CH, _D), jnp.float32),
        pltpu.VMEM((_CH, _D), jnp.float32),
        pltpu.SemaphoreType.DMA,
        pltpu.SemaphoreType.DMA,
        pltpu.SemaphoreType.DMA,
        pltpu.SemaphoreType.DMA,
        pltpu.SemaphoreType.DMA,
        pltpu.SemaphoreType.DMA,
        pltpu.SemaphoreType.DMA,
        pltpu.SemaphoreType.DMA,
        pltpu.SemaphoreType.DMA,
        pltpu.SemaphoreType.DMA,
    ],
)
def _sc_aggregate(src_hbm, dst_hbm, ev_hbm, ego_hbm, zeros_hbm, out_hbm,
                  acc, srcb0, srcb1, dstb0, dstb1, evb0, evb1,
                  rows0, rows1, sca0, sca1,
                  semg0, semg1, sems0, sems1, semsrc0, semsrc1,
                  semev0, semev1, semd0, semd1):
    c = lax.axis_index("c")
    s = lax.axis_index("s")
    wid = s * _NC + c

    rbase = s * _RPW

    def init_slice(base, n):
        @pl.when(c == 0)
        def _():
            pltpu.sync_copy(ego_hbm.at[pl.ds(base, n)],
                            acc.at[pl.ds(base, n)])

        @pl.when(c != 0)
        def _():
            pltpu.sync_copy(zeros_hbm.at[pl.ds(base, n)],
                            acc.at[pl.ds(base, n)])

    init_slice(rbase, _RPW)

    @pl.when(s == 0)
    def _():
        init_slice(_NS * _RPW, _TAIL)

    plsc.subcore_barrier()

    srcbs = (srcb0, srcb1)
    dstbs = (dstb0, dstb1)
    evbs = (evb0, evb1)
    rowss = (rows0, rows1)
    scas = (sca0, sca1)
    semg = (semg0, semg1)
    sems = (sems0, sems1)
    semsrc = (semsrc0, semsrc1)
    semev = (semev0, semev1)
    semd = (semd0, semd1)

    ebase = wid * _EPW

    def esl(i):
        return pl.ds(ebase + i * _CH, _CH)

    def start_src(i, b):
        pltpu.async_copy(src_hbm.at[esl(i)], srcbs[b], semsrc[b])

    def wait_src(b):
        pltpu.make_async_copy(src_hbm.at[esl(0)], srcbs[b], semsrc[b]).wait()

    def start_ev(i, b):
        pltpu.async_copy(ev_hbm.at[esl(i)], evbs[b], semev[b])

    def wait_ev(b):
        pltpu.make_async_copy(ev_hbm.at[esl(0)], evbs[b], semev[b]).wait()

    def start_dst(i, b):
        pltpu.async_copy(dst_hbm.at[esl(i)], dstbs[b], semd[b])

    def wait_dst(b):
        pltpu.make_async_copy(dst_hbm.at[esl(0)], dstbs[b], semd[b]).wait()

    def start_gather(b):
        pltpu.async_copy(ego_hbm.at[srcbs[b]], rowss[b], semg[b])

    def wait_gather(b):
        pltpu.make_async_copy(ego_hbm.at[srcbs[b]], rowss[b], semg[b]).wait()

    def start_scatter(b):
        pltpu.async_copy(scas[b], acc.at[dstbs[b]], sems[b], add=True)

    def wait_scatter(b):
        pltpu.make_async_copy(scas[b], acc.at[dstbs[b]], sems[b]).wait()

    def scale_chunk(b):
        rr = rowss[b]
        so = scas[b]
        evr = evbs[b]

        def body(g, carry):
            wvec = evr[pl.ds(g * 16, 16)]
            base = g * 16
            for l in range(16):
                w = lax.gather(
                    wvec, jnp.full((16, 1), l, jnp.int32),
                    lax.GatherDimensionNumbers(
                        offset_dims=(), collapsed_slice_dims=(0,),
                        start_index_map=(0,)),
                    (1,), mode=lax.GatherScatterMode.PROMISE_IN_BOUNDS)
                e = base + l
                for j in range(_D // 16):
                    sl = pl.ds(j * 16, 16)
                    so[e, sl] = rr[e, sl] * w
            return carry

        lax.fori_loop(0, _G, body, 0)

    def steady_chunk(i, b):
        b1 = b ^ 1
        wait_scatter(b)
        wait_gather(b)
        @pl.when(i + 2 < _NCHUNK)
        def _():
            start_src(i + 2, b)
        start_dst(i, b)
        wait_src(b1)
        start_gather(b1)
        wait_ev(b)
        scale_chunk(b)
        wait_dst(b)
        start_scatter(b)
        @pl.when(i + 2 < _NCHUNK)
        def _():
            start_ev(i + 2, b)

    pltpu.sync_copy(src_hbm.at[esl(0)], srcb0)
    pltpu.sync_copy(dst_hbm.at[esl(0)], dstb0)
    pltpu.sync_copy(ev_hbm.at[esl(0)], evb0)
    pltpu.sync_copy(src_hbm.at[esl(1)], srcb1)
    pltpu.sync_copy(dst_hbm.at[esl(1)], dstb1)
    pltpu.sync_copy(ev_hbm.at[esl(1)], evb1)
    start_gather(0)

    start_gather(1)
    wait_gather(0)
    start_src(2, 0)
    scale_chunk(0)
    start_scatter(0)
    start_ev(2, 0)
    wait_gather(1)
    start_src(3, 1)
    wait_src(0)
    start_gather(0)
    scale_chunk(1)
    start_scatter(1)
    start_ev(3, 1)

    def pair_body(k, carry):
        steady_chunk(2 * k, 0)
        steady_chunk(2 * k + 1, 1)
        return carry

    lax.fori_loop(1, (_NCHUNK - 1) // 2, pair_body, 0)

    wait_scatter(0)
    wait_gather(0)
    start_dst(_NCHUNK - 1, 0)
    wait_ev(0)
    scale_chunk(0)
    wait_dst(0)
    start_scatter(0)

    wait_scatter(1)
    wait_scatter(0)

    plsc.subcore_barrier()

    pltpu.sync_copy(acc.at[pl.ds(rbase, _RPW)],
                    out_hbm.at[c, pl.ds(rbase, _RPW)])

    @pl.when(s == 0)
    def _():
        pltpu.sync_copy(acc.at[pl.ds(_NS * _RPW, _TAIL)],
                        out_hbm.at[c, pl.ds(_NS * _RPW, _TAIL)])


def _tc_body(p_ref, wt_ref, b_ref, o_ref):
    x = p_ref[0] + p_ref[1]
    y = jnp.dot(x, wt_ref[...], preferred_element_type=jnp.float32)
    y = y + b_ref[...]
    o_ref[...] = jnp.where(y >= 0, y, y * jnp.float32(0.01))


_BR = 1000


def _tc_dense(partial, wt, b2):
    return pl.pallas_call(
        _tc_body,
        grid=(_N // _BR,),
        in_specs=[
            pl.BlockSpec((_NC, _BR, _D), lambda i: (0, i, 0)),
            pl.BlockSpec((_D, _D), lambda i: (0, 0)),
            pl.BlockSpec((1, _D), lambda i: (0, 0)),
        ],
        out_specs=pl.BlockSpec((_BR, _D), lambda i: (i, 0)),
        out_shape=jax.ShapeDtypeStruct((_N, _D), jnp.float32),
    )(partial, wt, b2)


@jax.jit
def kernel(edge_index, edge_values, ego_embeddings, W, b):
    src = edge_index[0]
    dst = edge_index[1]
    ev = edge_values
    zeros = jnp.zeros((_N, _D), jnp.float32)
    partial = _sc_aggregate(src, dst, ev, ego_embeddings, zeros)
    return _tc_dense(partial, W.T, b.reshape(1, _D))


# speedup vs baseline: 10.1754x; 1.0246x over previous
import functools

import jax
import jax.numpy as jnp
from jax import lax
from jax.experimental import pallas as pl
from jax.experimental.pallas import tpu as pltpu
from jax.experimental.pallas import tpu_sc as plsc

_N = 10000
_E = 320000
_D = 128

_NC = 2
_NS = 16
_NW = _NC * _NS
_EPW = _E // _NW
_CH = 80
_NCHUNK = _EPW // _CH
_G = _CH // 16
_RPW = 624
_TAIL = _N - _NS * _RPW

_mesh = plsc.VectorSubcoreMesh(core_axis_name="c", subcore_axis_name="s")


@functools.partial(
    pl.kernel,
    out_type=jax.ShapeDtypeStruct((_NC, _N, _D), jnp.float32),
    mesh=_mesh,
    scratch_types=[
        pltpu.VMEM_SHARED((_N, _D), jnp.float32),
        pltpu.VMEM((_CH,), jnp.int32),
        pltpu.VMEM((_CH,), jnp.int32),
        pltpu.VMEM((_CH,), jnp.int32),
        pltpu.VMEM((_CH,), jnp.int32),
        pltpu.VMEM((_CH,), jnp.float32),
        pltpu.VMEM((_CH,), jnp.float32),
        pltpu.VMEM((_CH, _D), jnp.float32),
        pltpu.VMEM((_CH, _D), jnp.float32),
        pltpu.VMEM((_CH, _D), jnp.float32),
        pltpu.VMEM((_CH, _D), jnp.float32),
        pltpu.SemaphoreType.DMA,
        pltpu.SemaphoreType.DMA,
        pltpu.SemaphoreType.DMA,
        pltpu.SemaphoreType.DMA,
        pltpu.SemaphoreType.DMA,
        pltpu.SemaphoreType.DMA,
        pltpu.SemaphoreType.DMA,
        pltpu.SemaphoreType.DMA,
        pltpu.SemaphoreType.DMA,
        pltpu.SemaphoreType.DMA,
    ],
)
def _sc_aggregate(src_hbm, dst_hbm, ev_hbm, ego_hbm, out_hbm,
                  acc, srcb0, srcb1, dstb0, dstb1, evb0, evb1,
                  rows0, rows1, sca0, sca1,
                  semg0, semg1, sems0, sems1, semsrc0, semsrc1,
                  semev0, semev1, semd0, semd1):
    c = lax.axis_index("c")
    s = lax.axis_index("s")
    wid = s * _NC + c

    rbase = s * _RPW

    @pl.when(c == 0)
    def _():
        pltpu.sync_copy(ego_hbm.at[pl.ds(rbase, _RPW)],
                        acc.at[pl.ds(rbase, _RPW)])

        @pl.when(s == 0)
        def _():
            pltpu.sync_copy(ego_hbm.at[pl.ds(_NS * _RPW, _TAIL)],
                            acc.at[pl.ds(_NS * _RPW, _TAIL)])

    @pl.when(c != 0)
    def _():
        zvec = jnp.zeros((16,), jnp.float32)

        def zbody(e, carry):
            for j in range(_D // 16):
                rows0[e, pl.ds(j * 16, 16)] = zvec
            return carry

        lax.fori_loop(0, _CH, zbody, 0)
        for k in range(7):
            pltpu.async_copy(rows0, acc.at[pl.ds(rbase + k * _CH, _CH)],
                             semg0)
        pltpu.async_copy(rows0.at[pl.ds(0, 64)],
                         acc.at[pl.ds(rbase + 7 * _CH, 64)], semg0)
        for k in range(7):
            pltpu.make_async_copy(
                rows0, acc.at[pl.ds(rbase + k * _CH, _CH)], semg0).wait()
        pltpu.make_async_copy(
            rows0.at[pl.ds(0, 64)],
            acc.at[pl.ds(rbase + 7 * _CH, 64)], semg0).wait()

        @pl.when(s == 0)
        def _():
            pltpu.sync_copy(rows0.at[pl.ds(0, _TAIL)],
                            acc.at[pl.ds(_NS * _RPW, _TAIL)])

    plsc.subcore_barrier()

    srcbs = (srcb0, srcb1)
    dstbs = (dstb0, dstb1)
    evbs = (evb0, evb1)
    rowss = (rows0, rows1)
    scas = (sca0, sca1)
    semg = (semg0, semg1)
    sems = (sems0, sems1)
    semsrc = (semsrc0, semsrc1)
    semev = (semev0, semev1)
    semd = (semd0, semd1)

    ebase = wid * _EPW

    def esl(i):
        return pl.ds(ebase + i * _CH, _CH)

    def start_src(i, b):
        pltpu.async_copy(src_hbm.at[esl(i)], srcbs[b], semsrc[b])

    def wait_src(b):
        pltpu.make_async_copy(src_hbm.at[esl(0)], srcbs[b], semsrc[b]).wait()

    def start_ev(i, b):
        pltpu.async_copy(ev_hbm.at[esl(i)], evbs[b], semev[b])

    def wait_ev(b):
        pltpu.make_async_copy(ev_hbm.at[esl(0)], evbs[b], semev[b]).wait()

    def start_dst(i, b):
        pltpu.async_copy(dst_hbm.at[esl(i)], dstbs[b], semd[b])

    def wait_dst(b):
        pltpu.make_async_copy(dst_hbm.at[esl(0)], dstbs[b], semd[b]).wait()

    def start_gather(b):
        pltpu.async_copy(ego_hbm.at[srcbs[b]], rowss[b], semg[b])

    def wait_gather(b):
        pltpu.make_async_copy(ego_hbm.at[srcbs[b]], rowss[b], semg[b]).wait()

    def start_scatter(b):
        pltpu.async_copy(scas[b], acc.at[dstbs[b]], sems[b], add=True)

    def wait_scatter(b):
        pltpu.make_async_copy(scas[b], acc.at[dstbs[b]], sems[b]).wait()

    def scale_chunk(b):
        rr = rowss[b]
        so = scas[b]
        evr = evbs[b]

        def body(g, carry):
            wvec = evr[pl.ds(g * 16, 16)]
            base = g * 16
            for l in range(16):
                w = lax.gather(
                    wvec, jnp.full((16, 1), l, jnp.int32),
                    lax.GatherDimensionNumbers(
                        offset_dims=(), collapsed_slice_dims=(0,),
                        start_index_map=(0,)),
                    (1,), mode=lax.GatherScatterMode.PROMISE_IN_BOUNDS)
                e = base + l
                for j in range(_D // 16):
                    sl = pl.ds(j * 16, 16)
                    so[e, sl] = rr[e, sl] * w
            return carry

        lax.fori_loop(0, _G, body, 0)

    def steady_chunk(i, b):
        b1 = b ^ 1
        wait_scatter(b)
        wait_gather(b)
        @pl.when(i + 2 < _NCHUNK)
        def _():
            start_src(i + 2, b)
        start_dst(i, b)
        wait_src(b1)
        start_gather(b1)
        wait_ev(b)
        scale_chunk(b)
        wait_dst(b)
        start_scatter(b)
        @pl.when(i + 2 < _NCHUNK)
        def _():
            start_ev(i + 2, b)

    pltpu.sync_copy(src_hbm.at[esl(0)], srcb0)
    pltpu.sync_copy(dst_hbm.at[esl(0)], dstb0)
    pltpu.sync_copy(ev_hbm.at[esl(0)], evb0)
    pltpu.sync_copy(src_hbm.at[esl(1)], srcb1)
    pltpu.sync_copy(dst_hbm.at[esl(1)], dstb1)
    pltpu.sync_copy(ev_hbm.at[esl(1)], evb1)
    start_gather(0)

    start_gather(1)
    wait_gather(0)
    start_src(2, 0)
    scale_chunk(0)
    start_scatter(0)
    start_ev(2, 0)
    wait_gather(1)
    start_src(3, 1)
    wait_src(0)
    start_gather(0)
    scale_chunk(1)
    start_scatter(1)
    start_ev(3, 1)

    def pair_body(k, carry):
        steady_chunk(2 * k, 0)
        steady_chunk(2 * k + 1, 1)
        return carry

    lax.fori_loop(1, (_NCHUNK - 1) // 2, pair_body, 0)

    wait_scatter(0)
    wait_gather(0)
    start_dst(_NCHUNK - 1, 0)
    wait_ev(0)
    scale_chunk(0)
    wait_dst(0)
    start_scatter(0)

    wait_scatter(1)
    wait_scatter(0)

    plsc.subcore_barrier()

    pltpu.sync_copy(acc.at[pl.ds(rbase, _RPW)],
                    out_hbm.at[c, pl.ds(rbase, _RPW)])

    @pl.when(s == 0)
    def _():
        pltpu.sync_copy(acc.at[pl.ds(_NS * _RPW, _TAIL)],
                        out_hbm.at[c, pl.ds(_NS * _RPW, _TAIL)])


def _tc_body(p_ref, wt_ref, b_ref, o_ref):
    x = p_ref[0] + p_ref[1]
    y = jnp.dot(x, wt_ref[...], preferred_element_type=jnp.float32)
    y = y + b_ref[...]
    o_ref[...] = jnp.where(y >= 0, y, y * jnp.float32(0.01))


_BR = 2000


def _tc_dense(partial, wt, b2):
    return pl.pallas_call(
        _tc_body,
        grid=(_N // _BR,),
        in_specs=[
            pl.BlockSpec((_NC, _BR, _D), lambda i: (0, i, 0)),
            pl.BlockSpec((_D, _D), lambda i: (0, 0)),
            pl.BlockSpec((1, _D), lambda i: (0, 0)),
        ],
        out_specs=pl.BlockSpec((_BR, _D), lambda i: (i, 0)),
        out_shape=jax.ShapeDtypeStruct((_N, _D), jnp.float32),
    )(partial, wt, b2)


@jax.jit
def kernel(edge_index, edge_values, ego_embeddings, W, b):
    src = edge_index[0]
    dst = edge_index[1]
    ev = edge_values
    partial = _sc_aggregate(src, dst, ev, ego_embeddings)
    return _tc_dense(partial, W.T, b.reshape(1, _D))


# speedup vs baseline: 10.6252x; 1.0442x over previous
import functools

import jax
import jax.numpy as jnp
from jax import lax
from jax.experimental import pallas as pl
from jax.experimental.pallas import tpu as pltpu
from jax.experimental.pallas import tpu_sc as plsc

_N = 10000
_E = 320000
_D = 128

_NC = 2
_NS = 16
_NW = _NC * _NS
_EPW = _E // _NW
_CH = 80
_NCHUNK = _EPW // _CH
_G = _CH // 16
_RPW = 624
_TAIL = _N - _NS * _RPW

_mesh = plsc.VectorSubcoreMesh(core_axis_name="c", subcore_axis_name="s")


@functools.partial(
    pl.kernel,
    out_type=jax.ShapeDtypeStruct((_NC, _N, _D), jnp.float32),
    mesh=_mesh,
    scratch_types=[
        pltpu.VMEM_SHARED((_N, _D), jnp.float32),
        pltpu.VMEM((_CH,), jnp.int32),
        pltpu.VMEM((_CH,), jnp.int32),
        pltpu.VMEM((_CH,), jnp.int32),
        pltpu.VMEM((_CH,), jnp.int32),
        pltpu.VMEM((_CH,), jnp.float32),
        pltpu.VMEM((_CH,), jnp.float32),
        pltpu.VMEM((_CH, _D), jnp.float32),
        pltpu.VMEM((_CH, _D), jnp.float32),
        pltpu.VMEM((_CH, _D), jnp.float32),
        pltpu.VMEM((_CH, _D), jnp.float32),
        pltpu.SemaphoreType.DMA,
        pltpu.SemaphoreType.DMA,
        pltpu.SemaphoreType.DMA,
        pltpu.SemaphoreType.DMA,
        pltpu.SemaphoreType.DMA,
        pltpu.SemaphoreType.DMA,
        pltpu.SemaphoreType.DMA,
        pltpu.SemaphoreType.DMA,
        pltpu.SemaphoreType.DMA,
        pltpu.SemaphoreType.DMA,
    ],
)
def _sc_aggregate(src_hbm, dst_hbm, ev_hbm, ego_hbm, out_hbm,
                  acc, srcb0, srcb1, dstb0, dstb1, evb0, evb1,
                  rows0, rows1, sca0, sca1,
                  semg0, semg1, sems0, sems1, semsrc0, semsrc1,
                  semev0, semev1, semd0, semd1):
    c = lax.axis_index("c")
    s = lax.axis_index("s")
    wid = s * _NC + c

    rbase = s * _RPW

    @pl.when(c == 0)
    def _():
        pltpu.sync_copy(ego_hbm.at[pl.ds(rbase, _RPW)],
                        acc.at[pl.ds(rbase, _RPW)])

        @pl.when(s == 0)
        def _():
            pltpu.sync_copy(ego_hbm.at[pl.ds(_NS * _RPW, _TAIL)],
                            acc.at[pl.ds(_NS * _RPW, _TAIL)])

    @pl.when(c != 0)
    def _():
        zvec = jnp.zeros((16,), jnp.float32)

        def zbody(e, carry):
            for j in range(_D // 16):
                rows0[e, pl.ds(j * 16, 16)] = zvec
            return carry

        lax.fori_loop(0, _CH, zbody, 0)
        for k in range(7):
            pltpu.async_copy(rows0, acc.at[pl.ds(rbase + k * _CH, _CH)],
                             semg0)
        pltpu.async_copy(rows0.at[pl.ds(0, 64)],
                         acc.at[pl.ds(rbase + 7 * _CH, 64)], semg0)
        for k in range(7):
            pltpu.make_async_copy(
                rows0, acc.at[pl.ds(rbase + k * _CH, _CH)], semg0).wait()
        pltpu.make_async_copy(
            rows0.at[pl.ds(0, 64)],
            acc.at[pl.ds(rbase + 7 * _CH, 64)], semg0).wait()

        @pl.when(s == 0)
        def _():
            pltpu.sync_copy(rows0.at[pl.ds(0, _TAIL)],
                            acc.at[pl.ds(_NS * _RPW, _TAIL)])

    plsc.subcore_barrier()

    srcbs = (srcb0, srcb1)
    dstbs = (dstb0, dstb1)
    evbs = (evb0, evb1)
    rowss = (rows0, rows1)
    scas = (sca0, sca1)
    semg = (semg0, semg1)
    sems = (sems0, sems1)
    semsrc = (semsrc0, semsrc1)
    semev = (semev0, semev1)
    semd = (semd0, semd1)

    ebase = wid * _EPW

    def esl(i):
        return pl.ds(ebase + i * _CH, _CH)

    def start_src(i, b):
        pltpu.async_copy(src_hbm.at[esl(i)], srcbs[b], semsrc[b])

    def wait_src(b):
        pltpu.make_async_copy(src_hbm.at[esl(0)], srcbs[b], semsrc[b]).wait()

    def start_ev(i, b):
        pltpu.async_copy(ev_hbm.at[esl(i)], evbs[b], semev[b])

    def wait_ev(b):
        pltpu.make_async_copy(ev_hbm.at[esl(0)], evbs[b], semev[b]).wait()

    def start_dst(i, b):
        pltpu.async_copy(dst_hbm.at[esl(i)], dstbs[b], semd[b])

    def wait_dst(b):
        pltpu.make_async_copy(dst_hbm.at[esl(0)], dstbs[b], semd[b]).wait()

    def start_gather(b):
        pltpu.async_copy(ego_hbm.at[srcbs[b]], rowss[b], semg[b])

    def wait_gather(b):
        pltpu.make_async_copy(ego_hbm.at[srcbs[b]], rowss[b], semg[b]).wait()

    def start_scatter(b):
        pltpu.async_copy(scas[b], acc.at[dstbs[b]], sems[b], add=True)

    def wait_scatter(b):
        pltpu.make_async_copy(scas[b], acc.at[dstbs[b]], sems[b]).wait()

    def scale_chunk(b):
        rr = rowss[b]
        so = scas[b]
        evr = evbs[b]

        def body(g, carry):
            wvec = evr[pl.ds(g * 16, 16)]
            base = g * 16
            for l in range(16):
                w = lax.gather(
                    wvec, jnp.full((16, 1), l, jnp.int32),
                    lax.GatherDimensionNumbers(
                        offset_dims=(), collapsed_slice_dims=(0,),
                        start_index_map=(0,)),
                    (1,), mode=lax.GatherScatterMode.PROMISE_IN_BOUNDS)
                e = base + l
                for j in range(_D // 16):
                    sl = pl.ds(j * 16, 16)
                    so[e, sl] = rr[e, sl] * w
            return carry

        lax.fori_loop(0, _G, body, 0)

    def steady_chunk(i, b):
        b1 = b ^ 1
        wait_scatter(b)
        wait_gather(b)
        @pl.when(i + 2 < _NCHUNK)
        def _():
            start_src(i + 2, b)
        start_dst(i, b)
        wait_src(b1)
        start_gather(b1)
        wait_ev(b)
        scale_chunk(b)
        wait_dst(b)
        start_scatter(b)
        @pl.when(i + 2 < _NCHUNK)
        def _():
            start_ev(i + 2, b)

    pltpu.sync_copy(src_hbm.at[esl(0)], srcb0)
    pltpu.sync_copy(dst_hbm.at[esl(0)], dstb0)
    pltpu.sync_copy(ev_hbm.at[esl(0)], evb0)
    pltpu.sync_copy(src_hbm.at[esl(1)], srcb1)
    pltpu.sync_copy(dst_hbm.at[esl(1)], dstb1)
    pltpu.sync_copy(ev_hbm.at[esl(1)], evb1)
    start_gather(0)

    start_gather(1)
    wait_gather(0)
    start_src(2, 0)
    scale_chunk(0)
    start_scatter(0)
    start_ev(2, 0)
    wait_gather(1)
    start_src(3, 1)
    wait_src(0)
    start_gather(0)
    scale_chunk(1)
    start_scatter(1)
    start_ev(3, 1)

    def pair_body(k, carry):
        steady_chunk(2 * k, 0)
        steady_chunk(2 * k + 1, 1)
        return carry

    lax.fori_loop(1, (_NCHUNK - 1) // 2, pair_body, 0)

    wait_scatter(0)
    wait_gather(0)
    start_dst(_NCHUNK - 1, 0)
    wait_ev(0)
    scale_chunk(0)
    wait_dst(0)
    start_scatter(0)

    wait_scatter(1)
    wait_scatter(0)

    plsc.subcore_barrier()

    pltpu.sync_copy(acc.at[pl.ds(rbase, _RPW)],
                    out_hbm.at[c, pl.ds(rbase, _RPW)])

    @pl.when(s == 0)
    def _():
        pltpu.sync_copy(acc.at[pl.ds(_NS * _RPW, _TAIL)],
                        out_hbm.at[c, pl.ds(_NS * _RPW, _TAIL)])


def _split_body(ei_ref, s_ref, d_ref):
    s_ref[...] = ei_ref[0]
    d_ref[...] = ei_ref[1]


def _tc_split(edge_index):
    return pl.pallas_call(
        _split_body,
        out_shape=[jax.ShapeDtypeStruct((_E,), jnp.int32),
                   jax.ShapeDtypeStruct((_E,), jnp.int32)],
    )(edge_index)


def _tc_body(p_ref, w_ref, b_ref, o_ref):
    x = p_ref[0] + p_ref[1]
    y = lax.dot_general(x, w_ref[...], (((1,), (1,)), ((), ())),
                        preferred_element_type=jnp.float32)
    y = y + b_ref[...]
    o_ref[...] = jnp.where(y >= 0, y, y * jnp.float32(0.01))


_BR = 2000


def _tc_dense(partial, w, b2):
    return pl.pallas_call(
        _tc_body,
        grid=(_N // _BR,),
        in_specs=[
            pl.BlockSpec((_NC, _BR, _D), lambda i: (0, i, 0)),
            pl.BlockSpec((_D, _D), lambda i: (0, 0)),
            pl.BlockSpec((1, _D), lambda i: (0, 0)),
        ],
        out_specs=pl.BlockSpec((_BR, _D), lambda i: (i, 0)),
        out_shape=jax.ShapeDtypeStruct((_N, _D), jnp.float32),
    )(partial, w, b2)


@jax.jit
def kernel(edge_index, edge_values, ego_embeddings, W, b):
    src, dst = _tc_split(edge_index)
    partial = _sc_aggregate(src, dst, edge_values, ego_embeddings)
    return _tc_dense(partial, W, b.reshape(1, _D))


# speedup vs baseline: 10.7438x; 1.0112x over previous
import functools

import jax
import jax.numpy as jnp
from jax import lax
from jax.experimental import pallas as pl
from jax.experimental.pallas import tpu as pltpu
from jax.experimental.pallas import tpu_sc as plsc

_N = 10000
_E = 320000
_D = 128

_NC = 2
_NS = 16
_NW = _NC * _NS
_EPW = _E // _NW
_CH = 80
_NCHUNK = _EPW // _CH
_G = _CH // 16
_RPW = 624
_TAIL = _N - _NS * _RPW

_mesh = plsc.VectorSubcoreMesh(core_axis_name="c", subcore_axis_name="s")


@functools.partial(
    pl.kernel,
    out_type=jax.ShapeDtypeStruct((_NC, _N, _D), jnp.float32),
    mesh=_mesh,
    scratch_types=[
        pltpu.VMEM_SHARED((_N, _D), jnp.float32),
        pltpu.VMEM((_CH,), jnp.int32),
        pltpu.VMEM((_CH,), jnp.int32),
        pltpu.VMEM((_CH,), jnp.int32),
        pltpu.VMEM((_CH,), jnp.int32),
        pltpu.VMEM((_CH,), jnp.float32),
        pltpu.VMEM((_CH,), jnp.float32),
        pltpu.VMEM((_CH, _D), jnp.float32),
        pltpu.VMEM((_CH, _D), jnp.float32),
        pltpu.VMEM((_CH, _D), jnp.float32),
        pltpu.VMEM((_CH, _D), jnp.float32),
        pltpu.SemaphoreType.DMA,
        pltpu.SemaphoreType.DMA,
        pltpu.SemaphoreType.DMA,
        pltpu.SemaphoreType.DMA,
        pltpu.SemaphoreType.DMA,
        pltpu.SemaphoreType.DMA,
        pltpu.SemaphoreType.DMA,
        pltpu.SemaphoreType.DMA,
        pltpu.SemaphoreType.DMA,
        pltpu.SemaphoreType.DMA,
    ],
)
def _sc_aggregate(src_hbm, dst_hbm, ev_hbm, ego_hbm, out_hbm,
                  acc, srcb0, srcb1, dstb0, dstb1, evb0, evb1,
                  rows0, rows1, sca0, sca1,
                  semg0, semg1, sems0, sems1, semsrc0, semsrc1,
                  semev0, semev1, semd0, semd1):
    c = lax.axis_index("c")
    s = lax.axis_index("s")
    wid = s * _NC + c

    rbase = s * _RPW

    srcbs = (srcb0, srcb1)
    dstbs = (dstb0, dstb1)
    evbs = (evb0, evb1)
    rowss = (rows0, rows1)
    scas = (sca0, sca1)
    semg = (semg0, semg1)
    sems = (sems0, sems1)
    semsrc = (semsrc0, semsrc1)
    semev = (semev0, semev1)
    semd = (semd0, semd1)

    ebase = wid * _EPW

    def esl(i):
        return pl.ds(ebase + i * _CH, _CH)

    def start_src(i, b):
        pltpu.async_copy(src_hbm.at[esl(i)], srcbs[b], semsrc[b])

    def wait_src(b):
        pltpu.make_async_copy(src_hbm.at[esl(0)], srcbs[b], semsrc[b]).wait()

    def start_ev(i, b):
        pltpu.async_copy(ev_hbm.at[esl(i)], evbs[b], semev[b])

    def wait_ev(b):
        pltpu.make_async_copy(ev_hbm.at[esl(0)], evbs[b], semev[b]).wait()

    def start_dst(i, b):
        pltpu.async_copy(dst_hbm.at[esl(i)], dstbs[b], semd[b])

    def wait_dst(b):
        pltpu.make_async_copy(dst_hbm.at[esl(0)], dstbs[b], semd[b]).wait()

    def start_gather(b):
        pltpu.async_copy(ego_hbm.at[srcbs[b]], rowss[b], semg[b])

    def wait_gather(b):
        pltpu.make_async_copy(ego_hbm.at[srcbs[b]], rowss[b], semg[b]).wait()

    def start_scatter(b):
        pltpu.async_copy(scas[b], acc.at[dstbs[b]], sems[b], add=True)

    def wait_scatter(b):
        pltpu.make_async_copy(scas[b], acc.at[dstbs[b]], sems[b]).wait()

    def scale_chunk(b):
        rr = rowss[b]
        so = scas[b]
        evr = evbs[b]

        def body(g, carry):
            wvec = evr[pl.ds(g * 16, 16)]
            base = g * 16
            for l in range(16):
                w = lax.gather(
                    wvec, jnp.full((16, 1), l, jnp.int32),
                    lax.GatherDimensionNumbers(
                        offset_dims=(), collapsed_slice_dims=(0,),
                        start_index_map=(0,)),
                    (1,), mode=lax.GatherScatterMode.PROMISE_IN_BOUNDS)
                e = base + l
                for j in range(_D // 16):
                    sl = pl.ds(j * 16, 16)
                    so[e, sl] = rr[e, sl] * w
            return carry

        lax.fori_loop(0, _G, body, 0)

    def steady_chunk(i, b):
        b1 = b ^ 1
        wait_scatter(b)
        wait_gather(b)
        @pl.when(i + 2 < _NCHUNK)
        def _():
            start_src(i + 2, b)
        start_dst(i, b)
        wait_src(b1)
        start_gather(b1)
        wait_ev(b)
        scale_chunk(b)
        wait_dst(b)
        start_scatter(b)
        @pl.when(i + 2 < _NCHUNK)
        def _():
            start_ev(i + 2, b)

    pltpu.sync_copy(src_hbm.at[esl(0)], srcb0)
    pltpu.sync_copy(dst_hbm.at[esl(0)], dstb0)
    pltpu.sync_copy(ev_hbm.at[esl(0)], evb0)
    pltpu.sync_copy(src_hbm.at[esl(1)], srcb1)
    pltpu.sync_copy(dst_hbm.at[esl(1)], dstb1)
    pltpu.sync_copy(ev_hbm.at[esl(1)], evb1)
    start_gather(0)
    start_gather(1)

    @pl.when(c == 0)
    def _():
        pltpu.sync_copy(ego_hbm.at[pl.ds(rbase, _RPW)],
                        acc.at[pl.ds(rbase, _RPW)])

        @pl.when(s == 0)
        def _():
            pltpu.sync_copy(ego_hbm.at[pl.ds(_NS * _RPW, _TAIL)],
                            acc.at[pl.ds(_NS * _RPW, _TAIL)])

    @pl.when(c != 0)
    def _():
        zvec = jnp.zeros((16,), jnp.float32)

        def zbody(e, carry):
            for j in range(_D // 16):
                sca0[e, pl.ds(j * 16, 16)] = zvec
            return carry

        lax.fori_loop(0, _CH, zbody, 0)
        for k in range(7):
            pltpu.async_copy(sca0, acc.at[pl.ds(rbase + k * _CH, _CH)],
                             sems0)
        pltpu.async_copy(sca0.at[pl.ds(0, 64)],
                         acc.at[pl.ds(rbase + 7 * _CH, 64)], sems0)
        for k in range(7):
            pltpu.make_async_copy(
                sca0, acc.at[pl.ds(rbase + k * _CH, _CH)], sems0).wait()
        pltpu.make_async_copy(
            sca0.at[pl.ds(0, 64)],
            acc.at[pl.ds(rbase + 7 * _CH, 64)], sems0).wait()

        @pl.when(s == 0)
        def _():
            pltpu.sync_copy(sca0.at[pl.ds(0, _TAIL)],
                            acc.at[pl.ds(_NS * _RPW, _TAIL)])

    plsc.subcore_barrier()

    wait_gather(0)
    start_src(2, 0)
    scale_chunk(0)
    start_scatter(0)
    start_ev(2, 0)
    wait_gather(1)
    start_src(3, 1)
    wait_src(0)
    start_gather(0)
    scale_chunk(1)
    start_scatter(1)
    start_ev(3, 1)

    def pair_body(k, carry):
        steady_chunk(2 * k, 0)
        steady_chunk(2 * k + 1, 1)
        return carry

    lax.fori_loop(1, (_NCHUNK - 1) // 2, pair_body, 0)

    wait_scatter(0)
    wait_gather(0)
    start_dst(_NCHUNK - 1, 0)
    wait_ev(0)
    scale_chunk(0)
    wait_dst(0)
    start_scatter(0)

    wait_scatter(1)
    wait_scatter(0)

    plsc.subcore_barrier()

    pltpu.sync_copy(acc.at[pl.ds(rbase, _RPW)],
                    out_hbm.at[c, pl.ds(rbase, _RPW)])

    @pl.when(s == 0)
    def _():
        pltpu.sync_copy(acc.at[pl.ds(_NS * _RPW, _TAIL)],
                        out_hbm.at[c, pl.ds(_NS * _RPW, _TAIL)])


def _split_body(ei_ref, s_ref, d_ref):
    s_ref[...] = ei_ref[0]
    d_ref[...] = ei_ref[1]


def _tc_split(edge_index):
    return pl.pallas_call(
        _split_body,
        out_shape=[jax.ShapeDtypeStruct((_E,), jnp.int32),
                   jax.ShapeDtypeStruct((_E,), jnp.int32)],
    )(edge_index)


def _tc_body(p_ref, w_ref, b_ref, o_ref):
    x = p_ref[0] + p_ref[1]
    y = lax.dot_general(x, w_ref[...], (((1,), (1,)), ((), ())),
                        preferred_element_type=jnp.float32)
    y = y + b_ref[...]
    o_ref[...] = jnp.where(y >= 0, y, y * jnp.float32(0.01))


_BR = 2000


def _tc_dense(partial, w, b2):
    return pl.pallas_call(
        _tc_body,
        grid=(_N // _BR,),
        in_specs=[
            pl.BlockSpec((_NC, _BR, _D), lambda i: (0, i, 0)),
            pl.BlockSpec((_D, _D), lambda i: (0, 0)),
            pl.BlockSpec((1, _D), lambda i: (0, 0)),
        ],
        out_specs=pl.BlockSpec((_BR, _D), lambda i: (i, 0)),
        out_shape=jax.ShapeDtypeStruct((_N, _D), jnp.float32),
    )(partial, w, b2)


@jax.jit
def kernel(edge_index, edge_values, ego_embeddings, W, b):
    src, dst = _tc_split(edge_index)
    partial = _sc_aggregate(src, dst, edge_values, ego_embeddings)
    return _tc_dense(partial, W, b.reshape(1, _D))
